# Initial kernel scaffold; baseline (speedup 1.0000x reference)
import jax
import jax.numpy as jnp
from jax.experimental import pallas as pl


def kernel(x, emb_id, table, W1, b1, W2, b2, W3, b3, W4, b4, W5, b5):
    raise NotImplementedError("write your pallas kernel here")


# speedup vs baseline: 1.9721x; 1.9721x over previous
import functools

import jax
import jax.numpy as jnp
from jax import lax
from jax.experimental import pallas as pl
from jax.experimental.pallas import tpu as pltpu
from jax.experimental.pallas import tpu_sc as plsc

_GATHER_CHUNK = 128


def _sc_gather(table, ids3):
    NW, n_ch, CH = ids3.shape
    D = table.shape[1]
    B = NW * n_ch * CH
    b_per_w = n_ch * CH
    NC = 2

    mesh = plsc.VectorSubcoreMesh(core_axis_name="c", subcore_axis_name="s")

    @functools.partial(
        pl.kernel,
        mesh=mesh,
        compiler_params=pltpu.CompilerParams(use_tc_tiling_on_sc=False),
        out_type=jax.ShapeDtypeStruct((B, D), jnp.float32),
        scratch_types=[
            pltpu.VMEM((n_ch, CH), jnp.int32),
            pltpu.VMEM((CH, D), jnp.float32),
            pltpu.SemaphoreType.DMA,
        ],
    )
    def gather_kernel(table_hbm, idx_hbm, out_hbm, idx_v, rows_v, sem):
        wid = lax.axis_index("s") * NC + lax.axis_index("c")
        base = wid * b_per_w
        pltpu.sync_copy(idx_hbm.at[wid], idx_v)

        def body(j, carry):
            pltpu.async_copy(table_hbm.at[idx_v.at[j]], rows_v, sem).wait()
            pltpu.sync_copy(rows_v, out_hbm.at[pl.ds(base + j * CH, CH)])
            return carry

        lax.fori_loop(0, n_ch, body, 0)

    return gather_kernel(table, ids3)


def _tc_mlp(x, emb, W1x, W1e, b1, W2, b2, W3, b3, W4, b4, W5, b5):
    N = x.shape[0]
    R = 2048
    grid = (N // R,)

    def body(x_ref, e_ref, w1x, w1e, bb1, w2, bb2, w3, bb3, w4, bb4, w5, bb5,
             out_ref):
        h = (jnp.dot(x_ref[...], w1x[...], preferred_element_type=jnp.float32)
             + jnp.dot(e_ref[...], w1e[...], preferred_element_type=jnp.float32)
             + bb1[...])
        h = jnp.maximum(h, 0.0)
        h = jnp.maximum(
            jnp.dot(h, w2[...], preferred_element_type=jnp.float32) + bb2[...], 0.0)
        h = jnp.maximum(
            jnp.dot(h, w3[...], preferred_element_type=jnp.float32) + bb3[...], 0.0)
        h = jnp.maximum(
            jnp.dot(h, w4[...], preferred_element_type=jnp.float32) + bb4[...], 0.0)
        o = jnp.dot(h, w5[...], preferred_element_type=jnp.float32) + bb5[...]
        out_ref[...] = jax.nn.sigmoid(o)

    full = lambda shape: pl.BlockSpec(shape, lambda i: (0,) * len(shape))
    return pl.pallas_call(
        body,
        grid=grid,
        in_specs=[
            pl.BlockSpec((R, x.shape[1]), lambda i: (i, 0)),
            pl.BlockSpec((R, emb.shape[1]), lambda i: (i, 0)),
            full(W1x.shape), full(W1e.shape), full(b1.shape),
            full(W2.shape), full(b2.shape),
            full(W3.shape), full(b3.shape),
            full(W4.shape), full(b4.shape),
            full(W5.shape), full(b5.shape),
        ],
        out_specs=pl.BlockSpec((R, W5.shape[1]), lambda i: (i, 0)),
        out_shape=jax.ShapeDtypeStruct((N, W5.shape[1]), jnp.float32),
    )(x, emb, W1x, W1e, b1, W2, b2, W3, b3, W4, b4, W5, b5)


def kernel(x, emb_id, table, W1, b1, W2, b2, W3, b3, W4, b4, W5, b5):
    N = x.shape[0]
    NW = 32
    CH = _GATHER_CHUNK
    n_ch = N // (NW * CH)
    ids3 = emb_id.astype(jnp.int32).reshape(NW, n_ch, CH)
    emb = _sc_gather(table, ids3)

    W1x = W1[: x.shape[1]]
    W1e = W1[x.shape[1]:]
    b2d = lambda b: b.reshape(1, -1)
    return _tc_mlp(x, emb, W1x, W1e, b2d(b1), W2, b2d(b2), W3, b2d(b3),
                   W4, b2d(b4), W5, b2d(b5))


# speedup vs baseline: 1.9848x; 1.0064x over previous
import functools

import jax
import jax.numpy as jnp
from jax import lax
from jax.experimental import pallas as pl
from jax.experimental.pallas import tpu as pltpu
from jax.experimental.pallas import tpu_sc as plsc

_GATHER_CHUNK = 128


def _sc_gather(table, ids3):
    NW, n_ch, CH = ids3.shape
    D = table.shape[1]
    B = NW * n_ch * CH
    b_per_w = n_ch * CH
    NC = 2

    mesh = plsc.VectorSubcoreMesh(core_axis_name="c", subcore_axis_name="s")

    @functools.partial(
        pl.kernel,
        mesh=mesh,
        compiler_params=pltpu.CompilerParams(use_tc_tiling_on_sc=False),
        out_type=jax.ShapeDtypeStruct((B, D), jnp.float32),
        scratch_types=[
            pltpu.VMEM((n_ch, CH), jnp.int32),
            pltpu.VMEM((CH, D), jnp.float32),
            pltpu.SemaphoreType.DMA,
        ],
    )
    def gather_kernel(table_hbm, idx_hbm, out_hbm, idx_v, rows_v, sem):
        wid = lax.axis_index("s") * NC + lax.axis_index("c")
        base = wid * b_per_w
        pltpu.sync_copy(idx_hbm.at[wid], idx_v)

        def body(j, carry):
            pltpu.async_copy(table_hbm.at[idx_v.at[j]], rows_v, sem).wait()
            pltpu.sync_copy(rows_v, out_hbm.at[pl.ds(base + j * CH, CH)])
            return carry

        lax.fori_loop(0, n_ch, body, 0)

    return gather_kernel(table, ids3)


def _tc_mlp(x, emb, W1x, W1e, b1, W2, b2, W3, b3, W4, b4, W5, b5):
    N = x.shape[0]
    R = 2048
    grid = (N // R,)

    bf = jnp.bfloat16

    def body(x_ref, e_ref, w1x, w1e, bb1, w2, bb2, w3, bb3, w4, bb4, w5, bb5,
             out_ref):
        f32 = jnp.float32
        h = (jnp.dot(x_ref[...].astype(bf), w1x[...],
                     preferred_element_type=f32)
             + jnp.dot(e_ref[...].astype(bf), w1e[...],
                       preferred_element_type=f32))
        h = jnp.maximum(h.astype(bf) + bb1[...], 0.0)
        h = jnp.maximum(
            jnp.dot(h, w2[...], preferred_element_type=f32).astype(bf)
            + bb2[...], 0.0)
        h = jnp.maximum(
            jnp.dot(h, w3[...], preferred_element_type=f32).astype(bf)
            + bb3[...], 0.0)
        h = jnp.maximum(
            jnp.dot(h, w4[...], preferred_element_type=f32).astype(bf)
            + bb4[...], 0.0)
        o = jnp.dot(h, w5[...], preferred_element_type=jnp.float32) + bb5[...]
        out_ref[...] = jax.nn.sigmoid(o)

    full = lambda shape: pl.BlockSpec(shape, lambda i: (0,) * len(shape))
    return pl.pallas_call(
        body,
        grid=grid,
        in_specs=[
            pl.BlockSpec((R, x.shape[1]), lambda i: (i, 0)),
            pl.BlockSpec((R, emb.shape[1]), lambda i: (i, 0)),
            full(W1x.shape), full(W1e.shape), full(b1.shape),
            full(W2.shape), full(b2.shape),
            full(W3.shape), full(b3.shape),
            full(W4.shape), full(b4.shape),
            full(W5.shape), full(b5.shape),
        ],
        out_specs=pl.BlockSpec((R, W5.shape[1]), lambda i: (i, 0)),
        out_shape=jax.ShapeDtypeStruct((N, W5.shape[1]), jnp.float32),
    )(x, emb, W1x, W1e, b1, W2, b2, W3, b3, W4, b4, W5, b5)


def kernel(x, emb_id, table, W1, b1, W2, b2, W3, b3, W4, b4, W5, b5):
    N = x.shape[0]
    NW = 32
    CH = _GATHER_CHUNK
    n_ch = N // (NW * CH)
    ids3 = emb_id.astype(jnp.int32).reshape(NW, n_ch, CH)
    emb = _sc_gather(table, ids3)

    bf = jnp.bfloat16
    W1x = W1[: x.shape[1]].astype(bf)
    W1e = W1[x.shape[1]:].astype(bf)
    b2d = lambda b: b.reshape(1, -1)
    bb = lambda b: b2d(b).astype(bf)
    return _tc_mlp(x, emb, W1x, W1e, bb(b1), W2.astype(bf), bb(b2),
                   W3.astype(bf), bb(b3), W4.astype(bf), bb(b4),
                   W5.astype(bf), b2d(b5))


# speedup vs baseline: 2.1516x; 1.0841x over previous
import functools

import jax
import jax.numpy as jnp
from jax import lax
from jax.experimental import pallas as pl
from jax.experimental.pallas import tpu as pltpu
from jax.experimental.pallas import tpu_sc as plsc

_GATHER_CHUNK = 128


def _sc_gather(table, ids3):
    NW, n_ch, CH = ids3.shape
    D = table.shape[1]
    B = NW * n_ch * CH
    b_per_w = n_ch * CH
    NC = 2

    mesh = plsc.VectorSubcoreMesh(core_axis_name="c", subcore_axis_name="s")

    @functools.partial(
        pl.kernel,
        mesh=mesh,
        out_type=jax.ShapeDtypeStruct((B, D), jnp.float32),
        scratch_types=[
            pltpu.VMEM((n_ch, CH), jnp.int32),
            pltpu.VMEM((CH, D), jnp.float32),
            pltpu.SemaphoreType.DMA,
        ],
    )
    def gather_kernel(table_hbm, idx_hbm, out_hbm, idx_v, rows_v, sem):
        wid = lax.axis_index("s") * NC + lax.axis_index("c")
        base = wid * b_per_w
        pltpu.sync_copy(idx_hbm.at[wid], idx_v)

        def body(j, carry):
            pltpu.async_copy(table_hbm.at[idx_v.at[j]], rows_v, sem).wait()
            pltpu.sync_copy(rows_v, out_hbm.at[pl.ds(base + j * CH, CH)])
            return carry

        lax.fori_loop(0, n_ch, body, 0)

    return gather_kernel(table, ids3)


def _tc_mlp(x, emb, W1x, W1e, b1, W2, b2, W3, b3, W4, b4, W5, b5):
    N = x.shape[0]
    R = 2048
    grid = (N // R,)

    bf = jnp.bfloat16

    def body(x_ref, e_ref, w1x, w1e, bb1, w2, bb2, w3, bb3, w4, bb4, w5, bb5,
             out_ref):
        f32 = jnp.float32
        h = (jnp.dot(x_ref[...].astype(bf), w1x[...],
                     preferred_element_type=f32)
             + jnp.dot(e_ref[...].astype(bf), w1e[...],
                       preferred_element_type=f32))
        h = jnp.maximum(h.astype(bf) + bb1[...], 0.0)
        h = jnp.maximum(
            jnp.dot(h, w2[...], preferred_element_type=f32).astype(bf)
            + bb2[...], 0.0)
        h = jnp.maximum(
            jnp.dot(h, w3[...], preferred_element_type=f32).astype(bf)
            + bb3[...], 0.0)
        h = jnp.maximum(
            jnp.dot(h, w4[...], preferred_element_type=f32).astype(bf)
            + bb4[...], 0.0)
        o = jnp.dot(h, w5[...], preferred_element_type=jnp.float32) + bb5[...]
        out_ref[...] = jax.nn.sigmoid(o)

    full = lambda shape: pl.BlockSpec(shape, lambda i: (0,) * len(shape))
    return pl.pallas_call(
        body,
        grid=grid,
        in_specs=[
            pl.BlockSpec((R, x.shape[1]), lambda i: (i, 0)),
            pl.BlockSpec((R, emb.shape[1]), lambda i: (i, 0)),
            full(W1x.shape), full(W1e.shape), full(b1.shape),
            full(W2.shape), full(b2.shape),
            full(W3.shape), full(b3.shape),
            full(W4.shape), full(b4.shape),
            full(W5.shape), full(b5.shape),
        ],
        out_specs=pl.BlockSpec((R, W5.shape[1]), lambda i: (i, 0)),
        out_shape=jax.ShapeDtypeStruct((N, W5.shape[1]), jnp.float32),
    )(x, emb, W1x, W1e, b1, W2, b2, W3, b3, W4, b4, W5, b5)


def kernel(x, emb_id, table, W1, b1, W2, b2, W3, b3, W4, b4, W5, b5):
    N = x.shape[0]
    NW = 32
    CH = _GATHER_CHUNK
    n_ch = N // (NW * CH)
    D = table.shape[1]
    DP = 128

    ids3 = emb_id.astype(jnp.int32).reshape(NW, n_ch, CH)
    table_p = jnp.pad(table, ((0, 0), (0, DP - D)))
    emb = _sc_gather(table_p, ids3)

    bf = jnp.bfloat16
    W1x = W1[: x.shape[1]].astype(bf)
    W1e = jnp.pad(W1[x.shape[1]:], ((0, DP - D), (0, 0))).astype(bf)
    b2d = lambda b: b.reshape(1, -1)
    bb = lambda b: b2d(b).astype(bf)
    return _tc_mlp(x, emb, W1x, W1e, bb(b1), W2.astype(bf), bb(b2),
                   W3.astype(bf), bb(b3), W4.astype(bf), bb(b4),
                   W5.astype(bf), b2d(b5))


# speedup vs baseline: 2.6334x; 1.2239x over previous
import functools

import jax
import jax.numpy as jnp
from jax import lax
from jax.experimental import pallas as pl
from jax.experimental.pallas import tpu as pltpu
from jax.experimental.pallas import tpu_sc as plsc

_GATHER_CHUNK = 128


def _sc_gather(table, ids3):
    NW, n_ch, CH = ids3.shape
    D = table.shape[1]
    B = NW * n_ch * CH
    b_per_w = n_ch * CH
    NC = 2

    mesh = plsc.VectorSubcoreMesh(core_axis_name="c", subcore_axis_name="s")

    @functools.partial(
        pl.kernel,
        mesh=mesh,
        out_type=jax.ShapeDtypeStruct((B, D), jnp.float32),
        scratch_types=[
            pltpu.VMEM((n_ch, CH), jnp.int32),
            pltpu.VMEM((CH, D), jnp.float32),
            pltpu.SemaphoreType.DMA,
        ],
    )
    def gather_kernel(table_hbm, idx_hbm, out_hbm, idx_v, rows_v, sem):
        wid = lax.axis_index("s") * NC + lax.axis_index("c")
        base = wid * b_per_w
        pltpu.sync_copy(idx_hbm.at[wid], idx_v)

        def body(j, carry):
            pltpu.async_copy(table_hbm.at[idx_v.at[j]], rows_v, sem).wait()
            pltpu.sync_copy(rows_v, out_hbm.at[pl.ds(base + j * CH, CH)])
            return carry

        lax.fori_loop(0, n_ch, body, 0)

    return gather_kernel(table, ids3)


def _tc_mlp(xT, emb, W1x, W1e, b1, W2, b2, W3, b3, W4, b4, W5, b5):
    N = xT.shape[1]
    R = 2048
    grid = (N // R,)

    bf = jnp.bfloat16

    def body(x_ref, e_ref, w1x, w1e, bb1, w2, bb2, w3, bb3, w4, bb4, w5, bb5,
             out_ref):
        f32 = jnp.float32
        hx = lax.dot_general(x_ref[...].astype(bf), w1x[...],
                             (((0,), (0,)), ((), ())),
                             preferred_element_type=f32)
        h = hx + jnp.dot(e_ref[...].astype(bf), w1e[...],
                         preferred_element_type=f32)
        h = jnp.maximum(h.astype(bf) + bb1[...], 0.0)
        h = jnp.maximum(
            jnp.dot(h, w2[...], preferred_element_type=f32).astype(bf)
            + bb2[...], 0.0)
        h = jnp.maximum(
            jnp.dot(h, w3[...], preferred_element_type=f32).astype(bf)
            + bb3[...], 0.0)
        h = jnp.maximum(
            jnp.dot(h, w4[...], preferred_element_type=f32).astype(bf)
            + bb4[...], 0.0)
        o = lax.dot_general(w5[...], h, (((0,), (1,)), ((), ())),
                            preferred_element_type=jnp.float32) + bb5[...]
        out_ref[...] = jax.nn.sigmoid(o)

    full = lambda shape: pl.BlockSpec(shape, lambda i: (0,) * len(shape))
    return pl.pallas_call(
        body,
        grid=grid,
        in_specs=[
            pl.BlockSpec((xT.shape[0], R), lambda i: (0, i)),
            pl.BlockSpec((R, emb.shape[1]), lambda i: (i, 0)),
            full(W1x.shape), full(W1e.shape), full(b1.shape),
            full(W2.shape), full(b2.shape),
            full(W3.shape), full(b3.shape),
            full(W4.shape), full(b4.shape),
            full(W5.shape), full(b5.shape),
        ],
        out_specs=pl.BlockSpec((W5.shape[1], R), lambda i: (0, i)),
        out_shape=jax.ShapeDtypeStruct((W5.shape[1], N), jnp.float32),
    )(xT, emb, W1x, W1e, b1, W2, b2, W3, b3, W4, b4, W5, b5)


def kernel(x, emb_id, table, W1, b1, W2, b2, W3, b3, W4, b4, W5, b5):
    N = x.shape[0]
    NW = 32
    CH = _GATHER_CHUNK
    n_ch = N // (NW * CH)
    D = table.shape[1]
    DP = 128

    ids3 = emb_id.astype(jnp.int32).reshape(NW, n_ch, CH)
    table_p = jnp.pad(table, ((0, 0), (0, DP - D)))
    emb = _sc_gather(table_p, ids3)

    bf = jnp.bfloat16
    W1x = W1[: x.shape[1]].astype(bf)
    W1e = jnp.pad(W1[x.shape[1]:], ((0, DP - D), (0, 0))).astype(bf)
    b2d = lambda b: b.reshape(1, -1)
    bb = lambda b: b2d(b).astype(bf)
    oT = _tc_mlp(x.T, emb, W1x, W1e, bb(b1), W2.astype(bf), bb(b2),
                 W3.astype(bf), bb(b3), W4.astype(bf), bb(b4),
                 W5.astype(bf), b5.reshape(-1, 1))
    return oT.T


# speedup vs baseline: 3.1309x; 1.1889x over previous
import functools

import jax
import jax.numpy as jnp
from jax import lax
from jax.experimental import pallas as pl
from jax.experimental.pallas import tpu as pltpu
from jax.experimental.pallas import tpu_sc as plsc

_GATHER_CHUNK = 128
_SEGMENTS = 4
_ROW_BLOCK = 2048


def _sc_gather(table, ids3):
    NW, n_ch, CH = ids3.shape
    D = table.shape[1]
    B = NW * n_ch * CH
    b_per_w = n_ch * CH
    NC = 2

    mesh = plsc.VectorSubcoreMesh(core_axis_name="c", subcore_axis_name="s")

    @functools.partial(
        pl.kernel,
        mesh=mesh,
        out_type=jax.ShapeDtypeStruct((B, D), jnp.float32),
        scratch_types=[
            pltpu.VMEM((n_ch, CH), jnp.int32),
            pltpu.VMEM((CH, D), jnp.float32),
            pltpu.VMEM((CH, D), jnp.float32),
            pltpu.SemaphoreType.DMA,
            pltpu.SemaphoreType.DMA,
        ],
    )
    def gather_kernel(table_hbm, idx_hbm, out_hbm, idx_v, rows_a, rows_b,
                      sem_a, sem_b):
        wid = lax.axis_index("s") * NC + lax.axis_index("c")
        base = wid * b_per_w
        pltpu.sync_copy(idx_hbm.at[wid], idx_v)

        bufs = (rows_a, rows_b)
        sems = (sem_a, sem_b)
        pltpu.async_copy(table_hbm.at[idx_v.at[0]], rows_a, sem_a)
        pltpu.async_copy(table_hbm.at[idx_v.at[1]], rows_b, sem_b)

        def body(i, carry):
            j0 = i * 2
            for b in range(2):
                j = j0 + b
                pltpu.make_async_copy(table_hbm.at[idx_v.at[j]], bufs[b],
                                      sems[b]).wait()
                pltpu.sync_copy(bufs[b], out_hbm.at[pl.ds(base + j * CH, CH)])

                @pl.when(j + 2 < n_ch)
                def _():
                    pltpu.async_copy(table_hbm.at[idx_v.at[j + 2]], bufs[b],
                                     sems[b])
            return carry

        lax.fori_loop(0, n_ch // 2, body, 0)

    return gather_kernel(table, ids3)


def _tc_mlp(xT, emb, W1x, W1e, b1, W2, b2, W3, b3, W4, b4, W5, b5):
    N = xT.shape[1]
    R = _ROW_BLOCK
    grid = (N // R,)

    bf = jnp.bfloat16

    def body(x_ref, e_ref, w1x, w1e, bb1, w2, bb2, w3, bb3, w4, bb4, w5, bb5,
             out_ref):
        f32 = jnp.float32
        hx = lax.dot_general(x_ref[...].astype(bf), w1x[...],
                             (((0,), (0,)), ((), ())),
                             preferred_element_type=f32)
        h = hx + jnp.dot(e_ref[...].astype(bf), w1e[...],
                         preferred_element_type=f32)
        h = jnp.maximum(h.astype(bf) + bb1[...], 0.0)
        h = jnp.maximum(
            jnp.dot(h, w2[...], preferred_element_type=f32).astype(bf)
            + bb2[...], 0.0)
        h = jnp.maximum(
            jnp.dot(h, w3[...], preferred_element_type=f32).astype(bf)
            + bb3[...], 0.0)
        h = jnp.maximum(
            jnp.dot(h, w4[...], preferred_element_type=f32).astype(bf)
            + bb4[...], 0.0)
        o = lax.dot_general(w5[...], h, (((0,), (1,)), ((), ())),
                            preferred_element_type=jnp.float32) + bb5[...]
        out_ref[...] = jax.nn.sigmoid(o)

    full = lambda shape: pl.BlockSpec(shape, lambda i: (0,) * len(shape))
    return pl.pallas_call(
        body,
        grid=grid,
        in_specs=[
            pl.BlockSpec((xT.shape[0], R), lambda i: (0, i)),
            pl.BlockSpec((R, emb.shape[1]), lambda i: (i, 0)),
            full(W1x.shape), full(W1e.shape), full(b1.shape),
            full(W2.shape), full(b2.shape),
            full(W3.shape), full(b3.shape),
            full(W4.shape), full(b4.shape),
            full(W5.shape), full(b5.shape),
        ],
        out_specs=pl.BlockSpec((W5.shape[1], R), lambda i: (0, i)),
        out_shape=jax.ShapeDtypeStruct((W5.shape[1], N), jnp.float32),
    )(xT, emb, W1x, W1e, b1, W2, b2, W3, b3, W4, b4, W5, b5)


def kernel(x, emb_id, table, W1, b1, W2, b2, W3, b3, W4, b4, W5, b5):
    N = x.shape[0]
    NW = 32
    CH = _GATHER_CHUNK
    S = _SEGMENTS
    seg = N // S
    n_ch = seg // (NW * CH)
    D = table.shape[1]
    DP = 128

    ids4 = emb_id.astype(jnp.int32).reshape(S, NW, n_ch, CH)
    table_p = jnp.pad(table, ((0, 0), (0, DP - D)))

    bf = jnp.bfloat16
    W1x = W1[: x.shape[1]].astype(bf)
    W1e = jnp.pad(W1[x.shape[1]:], ((0, DP - D), (0, 0))).astype(bf)
    b2d = lambda b: b.reshape(1, -1)
    bb = lambda b: b2d(b).astype(bf)
    args = (W1x, W1e, bb(b1), W2.astype(bf), bb(b2), W3.astype(bf), bb(b3),
            W4.astype(bf), bb(b4), W5.astype(bf), b5.reshape(-1, 1))

    xT = x.T
    outs = []
    for s in range(S):
        emb_s = _sc_gather(table_p, ids4[s])
        outs.append(_tc_mlp(lax.slice(xT, (0, s * seg), (2, (s + 1) * seg)),
                            emb_s, *args))
    return jnp.concatenate(outs, axis=1).T


# speedup vs baseline: 3.4062x; 1.0879x over previous
import functools

import jax
import jax.numpy as jnp
from jax import lax
from jax.experimental import pallas as pl
from jax.experimental.pallas import tpu as pltpu
from jax.experimental.pallas import tpu_sc as plsc

_GATHER_CHUNK = 128
_SEG_SIZES = (32768, 98304, 196608, 196608)
_ROW_BLOCK = 2048


def _sc_gather(table, ids3):
    NW, n_ch, CH = ids3.shape
    D = table.shape[1]
    B = NW * n_ch * CH
    b_per_w = n_ch * CH
    NC = 2

    mesh = plsc.VectorSubcoreMesh(core_axis_name="c", subcore_axis_name="s")

    @functools.partial(
        pl.kernel,
        mesh=mesh,
        out_type=jax.ShapeDtypeStruct((B, D), jnp.float32),
        scratch_types=[
            pltpu.VMEM((n_ch, CH), jnp.int32),
            pltpu.VMEM((CH, D), jnp.float32),
            pltpu.VMEM((CH, D), jnp.float32),
            pltpu.SemaphoreType.DMA,
            pltpu.SemaphoreType.DMA,
        ],
    )
    def gather_kernel(table_hbm, idx_hbm, out_hbm, idx_v, rows_a, rows_b,
                      sem_a, sem_b):
        wid = lax.axis_index("s") * NC + lax.axis_index("c")
        base = wid * b_per_w
        pltpu.sync_copy(idx_hbm.at[wid], idx_v)

        bufs = (rows_a, rows_b)
        sems = (sem_a, sem_b)
        pltpu.async_copy(table_hbm.at[idx_v.at[0]], rows_a, sem_a)
        pltpu.async_copy(table_hbm.at[idx_v.at[1]], rows_b, sem_b)

        def body(i, carry):
            j0 = i * 2
            for b in range(2):
                j = j0 + b
                pltpu.make_async_copy(table_hbm.at[idx_v.at[j]], bufs[b],
                                      sems[b]).wait()
                pltpu.sync_copy(bufs[b], out_hbm.at[pl.ds(base + j * CH, CH)])

                @pl.when(j + 2 < n_ch)
                def _():
                    pltpu.async_copy(table_hbm.at[idx_v.at[j + 2]], bufs[b],
                                     sems[b])
            return carry

        lax.fori_loop(0, n_ch // 2, body, 0)

    return gather_kernel(table, ids3)


def _tc_mlp(xT, emb, W1x, W1e, b1, W2, b2, W3, b3, W4, b4, W5, b5):
    N = xT.shape[1]
    R = _ROW_BLOCK
    grid = (N // R,)

    bf = jnp.bfloat16

    def body(x_ref, e_ref, w1x, w1e, bb1, w2, bb2, w3, bb3, w4, bb4, w5, bb5,
             out_ref):
        f32 = jnp.float32
        hx = lax.dot_general(x_ref[...].astype(bf), w1x[...],
                             (((0,), (0,)), ((), ())),
                             preferred_element_type=f32)
        h = hx + jnp.dot(e_ref[...].astype(bf), w1e[...],
                         preferred_element_type=f32)
        h = jnp.maximum(h.astype(bf) + bb1[...], 0.0)
        h = jnp.maximum(
            jnp.dot(h, w2[...], preferred_element_type=f32).astype(bf)
            + bb2[...], 0.0)
        h = jnp.maximum(
            jnp.dot(h, w3[...], preferred_element_type=f32).astype(bf)
            + bb3[...], 0.0)
        h = jnp.maximum(
            jnp.dot(h, w4[...], preferred_element_type=f32).astype(bf)
            + bb4[...], 0.0)
        o = lax.dot_general(w5[...], h, (((0,), (1,)), ((), ())),
                            preferred_element_type=jnp.float32) + bb5[...]
        out_ref[...] = jax.nn.sigmoid(o)

    full = lambda shape: pl.BlockSpec(shape, lambda i: (0,) * len(shape))
    return pl.pallas_call(
        body,
        grid=grid,
        in_specs=[
            pl.BlockSpec((xT.shape[0], R), lambda i: (0, i)),
            pl.BlockSpec((R, emb.shape[1]), lambda i: (i, 0)),
            full(W1x.shape), full(W1e.shape), full(b1.shape),
            full(W2.shape), full(b2.shape),
            full(W3.shape), full(b3.shape),
            full(W4.shape), full(b4.shape),
            full(W5.shape), full(b5.shape),
        ],
        out_specs=pl.BlockSpec((W5.shape[1], R), lambda i: (0, i)),
        out_shape=jax.ShapeDtypeStruct((W5.shape[1], N), jnp.float32),
    )(xT, emb, W1x, W1e, b1, W2, b2, W3, b3, W4, b4, W5, b5)


def kernel(x, emb_id, table, W1, b1, W2, b2, W3, b3, W4, b4, W5, b5):
    N = x.shape[0]
    NW = 32
    CH = _GATHER_CHUNK
    D = table.shape[1]
    DP = 128

    ids = emb_id.astype(jnp.int32)
    table_p = jnp.pad(table, ((0, 0), (0, DP - D)))

    bf = jnp.bfloat16
    W1x = W1[: x.shape[1]].astype(bf)
    W1e = jnp.pad(W1[x.shape[1]:], ((0, DP - D), (0, 0))).astype(bf)
    b2d = lambda b: b.reshape(1, -1)
    bb = lambda b: b2d(b).astype(bf)
    args = (W1x, W1e, bb(b1), W2.astype(bf), bb(b2), W3.astype(bf), bb(b3),
            W4.astype(bf), bb(b4), W5.astype(bf), b5.reshape(-1, 1))

    xT = x.T
    outs = []
    off = 0
    for seg in _SEG_SIZES:
        n_ch = seg // (NW * CH)
        ids3 = lax.slice(ids, (off,), (off + seg,)).reshape(NW, n_ch, CH)
        emb_s = _sc_gather(table_p, ids3)
        outs.append(_tc_mlp(lax.slice(xT, (0, off), (2, off + seg)),
                            emb_s, *args))
        off += seg

    out = jnp.zeros((W5.shape[1], N), jnp.float32)
    off = 0
    for seg, o in zip(_SEG_SIZES, outs):
        out = lax.dynamic_update_slice(out, o, (0, off))
        off += seg
    return out.T


# speedup vs baseline: 3.8523x; 1.1310x over previous
import functools

import jax
import jax.numpy as jnp
from jax import lax
from jax.experimental import pallas as pl
from jax.experimental.pallas import tpu as pltpu
from jax.experimental.pallas import tpu_sc as plsc

_GATHER_CHUNK = 128
_SEG_SIZES = (32768, 98304, 196608, 196608)
_ROW_BLOCK = 16384


def _sc_gather(table, ids3):
    NW, n_ch, CH = ids3.shape
    D = table.shape[1]
    B = NW * n_ch * CH
    b_per_w = n_ch * CH
    NC = 2

    mesh = plsc.VectorSubcoreMesh(core_axis_name="c", subcore_axis_name="s")

    @functools.partial(
        pl.kernel,
        mesh=mesh,
        out_type=jax.ShapeDtypeStruct((B, D), jnp.float32),
        scratch_types=[
            pltpu.VMEM((n_ch, CH), jnp.int32),
            pltpu.VMEM((CH, D), jnp.float32),
            pltpu.VMEM((CH, D), jnp.float32),
            pltpu.SemaphoreType.DMA,
            pltpu.SemaphoreType.DMA,
        ],
    )
    def gather_kernel(table_hbm, idx_hbm, out_hbm, idx_v, rows_a, rows_b,
                      sem_a, sem_b):
        wid = lax.axis_index("s") * NC + lax.axis_index("c")
        base = wid * b_per_w
        pltpu.sync_copy(idx_hbm.at[wid], idx_v)

        bufs = (rows_a, rows_b)
        sems = (sem_a, sem_b)
        pltpu.async_copy(table_hbm.at[idx_v.at[0]], rows_a, sem_a)
        pltpu.async_copy(table_hbm.at[idx_v.at[1]], rows_b, sem_b)

        def body(i, carry):
            j0 = i * 2
            for b in range(2):
                j = j0 + b
                pltpu.make_async_copy(table_hbm.at[idx_v.at[j]], bufs[b],
                                      sems[b]).wait()
                pltpu.sync_copy(bufs[b], out_hbm.at[pl.ds(base + j * CH, CH)])

                @pl.when(j + 2 < n_ch)
                def _():
                    pltpu.async_copy(table_hbm.at[idx_v.at[j + 2]], bufs[b],
                                     sems[b])
            return carry

        lax.fori_loop(0, n_ch // 2, body, 0)

    return gather_kernel(table, ids3)


def _tc_mlp(xT, emb, W1x, W1e, b1, W2, b2, W3, b3, W4, b4, W5, b5):
    N = xT.shape[1]
    R = _ROW_BLOCK
    grid = (N // R,)

    bf = jnp.bfloat16

    def body(x_ref, e_ref, w1x, w1e, bb1, w2, bb2, w3, bb3, w4, bb4, w5, bb5,
             out_ref):
        f32 = jnp.float32
        hx = lax.dot_general(x_ref[...].astype(bf), w1x[...],
                             (((0,), (0,)), ((), ())),
                             preferred_element_type=f32)
        h = hx + jnp.dot(e_ref[...].astype(bf), w1e[...],
                         preferred_element_type=f32)
        h = jnp.maximum(h.astype(bf) + bb1[...], 0.0)
        h = jnp.maximum(
            jnp.dot(h, w2[...], preferred_element_type=f32).astype(bf)
            + bb2[...], 0.0)
        h = jnp.maximum(
            jnp.dot(h, w3[...], preferred_element_type=f32).astype(bf)
            + bb3[...], 0.0)
        h = jnp.maximum(
            jnp.dot(h, w4[...], preferred_element_type=f32).astype(bf)
            + bb4[...], 0.0)
        o = lax.dot_general(w5[...], h, (((0,), (1,)), ((), ())),
                            preferred_element_type=jnp.float32) + bb5[...]
        out_ref[...] = jax.nn.sigmoid(o)

    full = lambda shape: pl.BlockSpec(shape, lambda i: (0,) * len(shape))
    return pl.pallas_call(
        body,
        grid=grid,
        in_specs=[
            pl.BlockSpec((xT.shape[0], R), lambda i: (0, i)),
            pl.BlockSpec((R, emb.shape[1]), lambda i: (i, 0)),
            full(W1x.shape), full(W1e.shape), full(b1.shape),
            full(W2.shape), full(b2.shape),
            full(W3.shape), full(b3.shape),
            full(W4.shape), full(b4.shape),
            full(W5.shape), full(b5.shape),
        ],
        out_specs=pl.BlockSpec((W5.shape[1], R), lambda i: (0, i)),
        out_shape=jax.ShapeDtypeStruct((W5.shape[1], N), jnp.float32),
    )(xT, emb, W1x, W1e, b1, W2, b2, W3, b3, W4, b4, W5, b5)


def kernel(x, emb_id, table, W1, b1, W2, b2, W3, b3, W4, b4, W5, b5):
    N = x.shape[0]
    NW = 32
    CH = _GATHER_CHUNK
    D = table.shape[1]
    DP = 128

    ids = emb_id.astype(jnp.int32)
    table_p = jnp.pad(table, ((0, 0), (0, DP - D)))

    bf = jnp.bfloat16
    W1x = W1[: x.shape[1]].astype(bf)
    W1e = jnp.pad(W1[x.shape[1]:], ((0, DP - D), (0, 0))).astype(bf)
    b2d = lambda b: b.reshape(1, -1)
    bb = lambda b: b2d(b).astype(bf)
    args = (W1x, W1e, bb(b1), W2.astype(bf), bb(b2), W3.astype(bf), bb(b3),
            W4.astype(bf), bb(b4), W5.astype(bf), b5.reshape(-1, 1))

    xT = x.T
    outs = []
    off = 0
    for seg in _SEG_SIZES:
        n_ch = seg // (NW * CH)
        ids3 = lax.slice(ids, (off,), (off + seg,)).reshape(NW, n_ch, CH)
        emb_s = _sc_gather(table_p, ids3)
        outs.append(_tc_mlp(lax.slice(xT, (0, off), (2, off + seg)),
                            emb_s, *args))
        off += seg

    out = jnp.zeros((W5.shape[1], N), jnp.float32)
    off = 0
    for seg, o in zip(_SEG_SIZES, outs):
        out = lax.dynamic_update_slice(out, o, (0, off))
        off += seg
    return out.T
